# Initial kernel scaffold; baseline (speedup 1.0000x reference)
import jax
import jax.numpy as jnp
from jax.experimental import pallas as pl


def kernel(x, edge_index, W1, b1, W2, b2, Wd1, bd1, Wd2, bd2):
    raise NotImplementedError("write your pallas kernel here")


# speedup vs baseline: 25.2129x; 25.2129x over previous
import functools

import jax
import jax.numpy as jnp
from jax import lax
from jax.experimental import pallas as pl
from jax.experimental.pallas import tpu as pltpu
from jax.experimental.pallas import tpu_sc as plsc

N = 10000
E = 320000
IN_DIM = 128
HID = 64
LAT = 16
ENC = 2 * LAT

NC, NS = 2, 16
NW = NC * NS
C = 100
NCH = 100
RPT = 632
NP = NS * RPT

def _deg_body(dst_hbm, ones_hbm, zeros_hbm, out_hbm, dst_v, ones_v, acc):
    c = lax.axis_index("c")
    s = lax.axis_index("s")
    wid = s * NC + c
    pltpu.sync_copy(dst_hbm.at[wid], dst_v)
    pltpu.sync_copy(ones_hbm, ones_v)
    pltpu.sync_copy(zeros_hbm, acc.at[pl.ds(s * RPT, RPT)])
    plsc.subcore_barrier()

    def chunk(j, carry):
        pltpu.sync_copy(ones_v, acc.at[dst_v.at[j]], add=True)
        return carry

    lax.fori_loop(0, NCH, chunk, 0)
    plsc.subcore_barrier()
    pltpu.sync_copy(acc.at[pl.ds(s * RPT, RPT)], out_hbm.at[c, pl.ds(s * RPT, RPT)])


@functools.cache
def _deg_kernel():
    return pl.kernel(
        _deg_body,
        out_type=jax.ShapeDtypeStruct((NC, NP, 16), jnp.float32),
        mesh=plsc.VectorSubcoreMesh(core_axis_name="c", subcore_axis_name="s"),
        compiler_params=pltpu.CompilerParams(use_tc_tiling_on_sc=False),
        scratch_types=[
            pltpu.VMEM((NCH, C), jnp.int32),
            pltpu.VMEM((C, 16), jnp.float32),
            pltpu.VMEM_SHARED((NP, 16), jnp.float32),
        ],
    )


@functools.cache
def _make_prop(d):
    def body(hs_hbm, src_hbm, dst_hbm, zeros_hbm, out_hbm,
             src_v, dst_v, rows_v, acc, sem):
        c = lax.axis_index("c")
        s = lax.axis_index("s")
        wid = s * NC + c
        pltpu.sync_copy(src_hbm.at[wid], src_v)
        pltpu.sync_copy(dst_hbm.at[wid], dst_v)
        pltpu.sync_copy(zeros_hbm, acc.at[pl.ds(s * RPT, RPT)])
        plsc.subcore_barrier()

        def chunk(j, carry):
            pltpu.async_copy(hs_hbm.at[src_v.at[j]], rows_v, sem).wait()
            pltpu.sync_copy(rows_v, acc.at[dst_v.at[j]], add=True)
            return carry

        lax.fori_loop(0, NCH, chunk, 0)
        plsc.subcore_barrier()
        pltpu.sync_copy(acc.at[pl.ds(s * RPT, RPT)],
                        out_hbm.at[c, pl.ds(s * RPT, RPT)])

    return pl.kernel(
        body,
        out_type=jax.ShapeDtypeStruct((NC, NP, d), jnp.float32),
        mesh=plsc.VectorSubcoreMesh(core_axis_name="c", subcore_axis_name="s"),
        compiler_params=pltpu.CompilerParams(use_tc_tiling_on_sc=False),
        scratch_types=[
            pltpu.VMEM((NCH, C), jnp.int32),
            pltpu.VMEM((NCH, C), jnp.int32),
            pltpu.VMEM((C, d), jnp.float32),
            pltpu.VMEM_SHARED((NP, d), jnp.float32),
            pltpu.SemaphoreType.DMA,
        ],
    )


def _dinv_of(d0_ref, d1_ref):
    deg = d0_ref[:, 0:1] + d1_ref[:, 0:1] + 1.0
    return lax.rsqrt(jnp.maximum(deg, 1.0))


def _tc_a_body(x_ref, w1_ref, d0_ref, d1_ref, o_ref):
    dinv = _dinv_of(d0_ref, d1_ref)
    h1 = jnp.dot(x_ref[...], w1_ref[...], preferred_element_type=jnp.float32)
    o_ref[...] = h1 * dinv


_tc_a = pl.pallas_call(
    _tc_a_body, out_shape=jax.ShapeDtypeStruct((N, HID), jnp.float32))


def _tc_b_body(p0_ref, p1_ref, h1s_ref, d0_ref, d1_ref, b1_ref, w2_ref, o_ref):
    dinv = _dinv_of(d0_ref, d1_ref)
    out1 = dinv * (p0_ref[...] + p1_ref[...] + h1s_ref[...]) + b1_ref[...]
    out1 = jnp.maximum(out1, 0.0)
    h2 = jnp.dot(out1, w2_ref[...], preferred_element_type=jnp.float32)
    o_ref[...] = h2 * dinv


_tc_b = pl.pallas_call(
    _tc_b_body, out_shape=jax.ShapeDtypeStruct((N, ENC), jnp.float32))


def _tc_c_body(p0_ref, p1_ref, h2s_ref, d0_ref, d1_ref, b2_ref, eps_ref,
               wd1_ref, bd1_ref, wd2_ref, bd2_ref,
               dec_ref, mu_ref, lv_ref):
    dinv = _dinv_of(d0_ref, d1_ref)
    enc = dinv * (p0_ref[...] + p1_ref[...] + h2s_ref[...]) + b2_ref[...]
    mu = enc[:, :LAT]
    lv = enc[:, LAT:]
    std = jnp.exp(0.5 * lv)
    z = mu + eps_ref[...] * std
    dd = jnp.dot(z, wd1_ref[...], preferred_element_type=jnp.float32)
    dd = jnp.maximum(dd + bd1_ref[...], 0.0)
    dec = jnp.dot(dd, wd2_ref[...], preferred_element_type=jnp.float32)
    dec_ref[...] = jax.nn.sigmoid(dec + bd2_ref[...])
    mu_ref[...] = mu
    lv_ref[...] = lv


_tc_c = pl.pallas_call(
    _tc_c_body,
    out_shape=[
        jax.ShapeDtypeStruct((N, IN_DIM), jnp.float32),
        jax.ShapeDtypeStruct((N, LAT), jnp.float32),
        jax.ShapeDtypeStruct((N, LAT), jnp.float32),
    ],
)


def kernel(x, edge_index, W1, b1, W2, b2, Wd1, bd1, Wd2, bd2):
    src_w = edge_index[0].astype(jnp.int32).reshape(NW, NCH, C)
    dst_w = edge_index[1].astype(jnp.int32).reshape(NW, NCH, C)

    ones16 = jnp.ones((C, 16), jnp.float32)
    z16 = jnp.zeros((RPT, 16), jnp.float32)
    z64 = jnp.zeros((RPT, HID), jnp.float32)
    z32 = jnp.zeros((RPT, ENC), jnp.float32)

    degp = _deg_kernel()(dst_w, ones16, z16)
    d0, d1 = degp[0, :N], degp[1, :N]

    h1s = _tc_a(x, W1, d0, d1)
    p1 = _make_prop(HID)(h1s, src_w, dst_w, z64)
    h2s = _tc_b(p1[0, :N], p1[1, :N], h1s, d0, d1, b1.reshape(1, HID), W2)
    p2 = _make_prop(ENC)(h2s, src_w, dst_w, z32)

    eps = jax.random.normal(jax.random.key(42), (N, LAT), jnp.float32)
    dec, mu, lv = _tc_c(p2[0, :N], p2[1, :N], h2s, d0, d1, b2.reshape(1, ENC), eps,
                        Wd1, bd1.reshape(1, HID), Wd2, bd2.reshape(1, IN_DIM))
    return (dec, mu, lv)
